# baseline (speedup 1.0000x reference)
import functools

import jax
import jax.numpy as jnp
from jax import lax
from jax.experimental import pallas as pl
from jax.experimental.pallas import tpu as pltpu
from jax.experimental.pallas import tpu_sc as plsc

_VOCAB = 1000000
_VPAD = 1000064
_SROWS = _VPAD // 4
_EMBED = 32
_BATCH = 4096
_HIST = 200

_NC = 2
_NS = 16
_NW = _NC * _NS
_ROWS_W = _BATCH // _NW
_PER_W = _ROWS_W * _HIST
_CH = _ROWS_W
_NCHUNK = _HIST
_NITER = _NCHUNK // 2

_COLS_W = 244
_PAIRS = _COLS_W // 2


def _sc_detranspose(table_t):
  mesh = plsc.VectorSubcoreMesh(core_axis_name="c", subcore_axis_name="s")

  @functools.partial(
      pl.kernel,
      mesh=mesh,
      out_type=jax.ShapeDtypeStruct((_SROWS, 128), jnp.float32),
      scratch_types=[
          pltpu.VMEM((2, _EMBED, 128), jnp.float32),
          pltpu.VMEM((2, _EMBED, 128), jnp.float32),
          pltpu.VMEM((_EMBED, 16), jnp.int32),
          pltpu.VMEM((_EMBED, 16), jnp.int32),
          pltpu.SemaphoreType.DMA,
          pltpu.SemaphoreType.DMA,
          pltpu.SemaphoreType.DMA,
          pltpu.SemaphoreType.DMA,
      ],
      compiler_params=pltpu.CompilerParams(
          use_tc_tiling_on_sc=True, needs_layout_passes=False),
  )
  def ka(tt_hbm, out_hbm, tin_v, tout_v, pc_fd, pc_c, gi0, gi1, so0, so1):
    wid = lax.axis_index("s") * _NC + lax.axis_index("c")
    col0 = wid * _COLS_W
    gsem = (gi0, gi1)
    ssem = (so0, so1)
    iota = lax.iota(jnp.int32, 16)

    for h in range(2):
      for d in range(16):
        fd = 16 * h + ((iota + d) & 15)
        pc_fd[16 * h + d, :] = fd
        pc_c[16 * h + d, :] = ((iota & 3) << 5) + fd

    def in_start(c, b):
      pltpu.async_copy(
          tt_hbm.at[:, pl.ds(c * 128, 128)], tin_v.at[b], gsem[b])

    def in_wait(b):
      pltpu.make_async_copy(
          tt_hbm.at[:, pl.ds(0, 128)], tin_v.at[b], gsem[b]).wait()

    def out_start(c, b):
      pltpu.async_copy(
          tout_v.at[b], out_hbm.at[pl.ds(c * _EMBED, _EMBED)], ssem[b])

    def out_wait(b):
      pltpu.make_async_copy(
          tout_v.at[b], out_hbm.at[pl.ds(0, _EMBED)], ssem[b]).wait()

    def transpose(b):
      for g2 in range(8):
        src_c = 16 * g2 + iota
        dst_r = 4 * g2 + (iota >> 2)
        for h in range(2):
          for d in range(16):
            fd = pc_fd[16 * h + d, :]
            cc = pc_c[16 * h + d, :]
            vals = plsc.load_gather(tin_v.at[b], [fd, src_c])
            plsc.store_scatter(tout_v.at[b], [dst_r, cc], vals)

    in_start(col0, 0)
    in_start(col0 + 1, 1)

    def body(i, carry):
      for j in range(2):
        c = col0 + i * 2 + j

        @pl.when(i > 0)
        def _():
          out_wait(j)

        in_wait(j)
        transpose(j)
        out_start(c, j)

        @pl.when(i < _PAIRS - 1)
        def _():
          in_start(c + 2, j)

      return carry

    lax.fori_loop(0, _PAIRS, body, 0)
    out_wait(0)
    out_wait(1)

    @pl.when(wid < 4)
    def _():
      c = 7808 + wid
      in_start(c, 0)
      in_wait(0)
      transpose(0)
      out_start(c, 0)
      out_wait(0)

  return ka(table_t)


def _sc_gather(idx_flat, scratch):
  mesh = plsc.VectorSubcoreMesh(core_axis_name="c", subcore_axis_name="s")

  @functools.partial(
      pl.kernel,
      mesh=mesh,
      out_type=jax.ShapeDtypeStruct((_HIST, _EMBED, _BATCH), jnp.float32),
      scratch_types=[
          pltpu.VMEM((_PER_W,), jnp.int32),
          pltpu.VMEM((_PER_W,), jnp.int32),
          pltpu.VMEM((_PER_W,), jnp.int32),
          pltpu.VMEM((2, _CH, 128), jnp.float32),
          pltpu.VMEM((2, _EMBED, _ROWS_W), jnp.float32),
          pltpu.VMEM((_EMBED, 16), jnp.int32),
          pltpu.SemaphoreType.DMA,
          pltpu.SemaphoreType.DMA,
          pltpu.SemaphoreType.DMA,
          pltpu.SemaphoreType.DMA,
      ],
      compiler_params=pltpu.CompilerParams(
          use_tc_tiling_on_sc=True, needs_layout_passes=False),
  )
  def kb(idx_hbm, table_hbm, out_hbm, idx_v, idxq_v, idxr_v, rows_v, trows_v,
         pc_fd, g0, g1, s0, s1):
    wid = lax.axis_index("s") * _NC + lax.axis_index("c")
    c0 = wid * _ROWS_W
    gsem = (g0, g1)
    ssem = (s0, s1)
    iota = lax.iota(jnp.int32, 16)

    for h in range(2):
      for d in range(16):
        pc_fd[16 * h + d, :] = 16 * h + ((iota + d) & 15)

    pltpu.sync_copy(idx_hbm.at[pl.ds(c0 * _HIST, _PER_W)], idx_v)

    def build_idxt(h, carry):
      for g in range(_ROWS_W // 16):
        flat = (iota + 16 * g) * _HIST + h
        v = plsc.load_gather(idx_v, [flat])
        pos = h * _ROWS_W + 16 * g
        idxq_v[pl.ds(pos, 16)] = v >> 2
        idxr_v[pl.ds(pos, 16)] = (v & 3) * _EMBED
      return carry

    lax.fori_loop(0, _HIST, build_idxt, 0)

    def gather_start(s, b):
      pltpu.async_copy(
          table_hbm.at[idxq_v.at[pl.ds(s * _CH, _CH)]],
          rows_v.at[b], gsem[b])

    def gather_wait(b):
      pltpu.make_async_copy(
          table_hbm.at[pl.ds(0, _CH)], rows_v.at[b], gsem[b]).wait()

    def store_start(s, b):
      pltpu.async_copy(
          trows_v.at[b], out_hbm.at[s, :, pl.ds(c0, _ROWS_W)], ssem[b])

    def store_wait(b):
      pltpu.make_async_copy(
          trows_v.at[b], out_hbm.at[0, :, pl.ds(c0, _ROWS_W)], ssem[b]).wait()

    gather_start(0, 0)
    gather_start(1, 1)

    def body(ho, carry):
      for j in range(2):
        s = ho * 2 + j

        @pl.when(ho > 0)
        def _():
          store_wait(j)

        gather_wait(j)
        for g in range(_ROWS_W // 16):
          src_r = 16 * g + iota
          rem16 = idxr_v[pl.ds(s * _CH + 16 * g, 16)]
          for h in range(2):
            for d in range(16):
              fd = pc_fd[16 * h + d, :]
              vals = plsc.load_gather(rows_v.at[j], [src_r, rem16 + fd])
              plsc.store_scatter(trows_v.at[j], [fd, src_r], vals)
        store_start(s, j)

        @pl.when(ho < _NITER - 1)
        def _():
          gather_start(s + 2, j)

      return carry

    lax.fori_loop(0, _NITER, body, 0)
    store_wait(0)
    store_wait(1)

  return kb(idx_flat, scratch)


def kernel(inputs, embeddings):
  idx_flat = inputs.reshape(-1).astype(jnp.int32)
  scratch = _sc_detranspose(embeddings.T)
  tail = lax.dynamic_slice(embeddings, (999936, 0), (64, _EMBED))
  scratch = lax.dynamic_update_slice(
      scratch, tail.reshape(16, 128), (249984, 0))
  out_t = _sc_gather(idx_flat, scratch)
  return out_t.transpose(2, 0, 1)


# speedup vs baseline: 1.4058x; 1.4058x over previous
import functools

import jax
import jax.numpy as jnp
from jax import lax
from jax.experimental import pallas as pl
from jax.experimental.pallas import tpu as pltpu
from jax.experimental.pallas import tpu_sc as plsc

_VOCAB = 1000000
_EMBED = 32
_BATCH = 4096
_HIST = 200
_TOTAL = _BATCH * _HIST

_NC = 2
_NS = 16
_NW = _NC * _NS
_PER_W = _TOTAL // _NW
_CHUNK = 800
_DEPTH = 4
_AHEAD = 2
_NCHUNKS = _PER_W // _CHUNK


def _sc_gather(idx_flat, table):
  mesh = plsc.VectorSubcoreMesh(core_axis_name="c", subcore_axis_name="s")

  @functools.partial(
      pl.kernel,
      mesh=mesh,
      out_type=jax.ShapeDtypeStruct((_TOTAL, _EMBED), jnp.float32),
      scratch_types=(
          [
              pltpu.VMEM((_PER_W,), jnp.int32),
              pltpu.VMEM((_DEPTH, _CHUNK, _EMBED), jnp.float32),
          ]
          + [pltpu.SemaphoreType.DMA] * (2 * _DEPTH)
      ),
      compiler_params=pltpu.CompilerParams(use_tc_tiling_on_sc=False),
  )
  def k(idx_hbm, table_hbm, out_hbm, idx_v, rows_v, *sems):
    wid = lax.axis_index("s") * _NC + lax.axis_index("c")
    base = wid * _PER_W
    gsem = sems[:_DEPTH]
    ssem = sems[_DEPTH:]

    pltpu.sync_copy(idx_hbm.at[pl.ds(base, _PER_W)], idx_v)

    def gather_start(i, b):
      return pltpu.async_copy(
          table_hbm.at[idx_v.at[pl.ds(i * _CHUNK, _CHUNK)]],
          rows_v.at[b], gsem[b])

    def store_start(i, b):
      return pltpu.async_copy(
          rows_v.at[b], out_hbm.at[pl.ds(base + i * _CHUNK, _CHUNK)],
          ssem[b])

    gathers = [None] * _DEPTH
    stores = [None] * _DEPTH
    for j in range(_AHEAD):
      gathers[j % _DEPTH] = gather_start(j, j % _DEPTH)
    for i in range(_NCHUNKS):
      b = i % _DEPTH
      nxt = i + _AHEAD
      if nxt < _NCHUNKS:
        nb = nxt % _DEPTH
        if stores[nb] is not None:
          stores[nb].wait()
          stores[nb] = None
        gathers[nb] = gather_start(nxt, nb)
      gathers[b].wait()
      stores[b] = store_start(i, b)
    for b in range(_DEPTH):
      if stores[b] is not None:
        stores[b].wait()

  return k(idx_flat, table)


def kernel(inputs, embeddings):
  idx_flat = inputs.reshape(-1).astype(jnp.int32)
  out = _sc_gather(idx_flat, embeddings)
  return out.reshape(_BATCH, _HIST, _EMBED)
